# baseline (speedup 1.0000x reference)
import functools

import jax
import jax.numpy as jnp
from jax import lax
from jax.experimental import pallas as pl
from jax.experimental.pallas import tpu as pltpu
from jax.experimental.pallas import tpu_sc as plsc

NC = 2
NS = 16
K = 128
NB = 3


def _mesh():
    return plsc.VectorSubcoreMesh(core_axis_name="c", subcore_axis_name="s")


def _make_hist(N, NP, ept, F=128):
    nch = ept // K
    rpt = NP // NS
    assert nch % NB == 0 and nch >= 2 * NB

    @functools.partial(
        pl.kernel,
        out_type=jax.ShapeDtypeStruct((NC, NP, F), jnp.float32),
        mesh=_mesh(),
        scratch_types=[
            pltpu.VMEM((NB, K), jnp.int32),
            pltpu.VMEM((K, F), jnp.float32),
            pltpu.VMEM_SHARED((NP, F), jnp.float32),
        ] + [pltpu.SemaphoreType.DMA] * (2 * NB),
    )
    def hist(dst_hbm, ones_hbm, zeros_hbm, out_hbm, dstc, ones_v, acc, *sems):
        si = sems[:NB]
        ss = sems[NB:]
        cid = lax.axis_index("c")
        sid = lax.axis_index("s")
        base = (cid * NS + sid) * ept
        pltpu.sync_copy(zeros_hbm.at[pl.ds(sid * rpt, rpt)],
                        acc.at[pl.ds(sid * rpt, rpt)])
        pltpu.sync_copy(ones_hbm, ones_v)
        plsc.subcore_barrier()

        def idx_cp(j, b):
            return pltpu.make_async_copy(
                dst_hbm.at[pl.ds(base + j * K, K)], dstc.at[b], si[b])

        def sc_cp(b):
            return pltpu.make_async_copy(ones_v, acc.at[dstc.at[b]], ss[b])

        def step(j, b, issue):
            nb = (b + 1) % NB
            idx_cp(j, b).wait()
            pltpu.async_copy(ones_v, acc.at[dstc.at[b]], ss[b], add=True)
            if issue:
                sc_cp(nb).wait()
                idx_cp(j + 1, nb).start()

        for b in range(NB):
            idx_cp(b, b).start()
        for b in range(NB):
            step(b, b, b == NB - 1)

        def body(jj, carry):
            j = NB * jj
            for b in range(NB):
                step(j + b, b, True)
            return carry

        lax.fori_loop(1, nch // NB - 1, body, 0)
        for b in range(NB):
            step(nch - NB + b, b, b < NB - 1)
        for b in range(NB):
            sc_cp(b).wait()
        plsc.subcore_barrier()
        pltpu.sync_copy(acc.at[pl.ds(sid * rpt, rpt)],
                        out_hbm.at[cid].at[pl.ds(sid * rpt, rpt)])

    return hist


def _make_prop(N, NP, src_stride, dst_stride, ept, F=128):
    nch = ept // K
    rpt = NP // NS
    assert nch % NB == 0 and nch >= 2 * NB

    @functools.partial(
        pl.kernel,
        out_type=jax.ShapeDtypeStruct((NC, NP, F), jnp.float32),
        mesh=_mesh(),
        scratch_types=[
            pltpu.VMEM((NB, K), jnp.int32),
            pltpu.VMEM((NB, K), jnp.int32),
            pltpu.VMEM((NB, K, F), jnp.float32),
            pltpu.VMEM_SHARED((NP, F), jnp.float32),
        ] + [pltpu.SemaphoreType.DMA] * (3 * NB),
    )
    def prop(table_hbm, src_hbm, dst_hbm, zeros_hbm, out_hbm,
             srcc, dstc, rows, acc, *sems):
        si = sems[:NB]
        sg = sems[NB:2 * NB]
        ss = sems[2 * NB:]
        cid = lax.axis_index("c")
        sid = lax.axis_index("s")
        base_src = cid * src_stride + sid * ept
        base_dst = cid * dst_stride + sid * ept
        pltpu.sync_copy(zeros_hbm.at[pl.ds(sid * rpt, rpt)],
                        acc.at[pl.ds(sid * rpt, rpt)])
        plsc.subcore_barrier()

        def idx_cp(j, b):
            return (
                pltpu.make_async_copy(
                    src_hbm.at[pl.ds(base_src + j * K, K)], srcc.at[b], si[b]),
                pltpu.make_async_copy(
                    dst_hbm.at[pl.ds(base_dst + j * K, K)], dstc.at[b], si[b]))

        def g_cp(b):
            return pltpu.make_async_copy(table_hbm.at[srcc.at[b]],
                                         rows.at[b], sg[b])

        def sc_cp(b):
            return pltpu.make_async_copy(rows.at[b], acc.at[dstc.at[b]], ss[b])

        def step(j, b, scatter, issue):
            p2 = (b - 2) % NB
            nb = (b + 1) % NB
            for c in idx_cp(j, b):
                c.wait()
            g_cp(b).start()
            if scatter:
                g_cp(p2).wait()
                pltpu.async_copy(rows.at[p2], acc.at[dstc.at[p2]], ss[p2],
                                 add=True)
            if issue:
                sc_cp(nb).wait()
                for c in idx_cp(j + 1, nb):
                    c.start()

        for b in range(NB):
            for c in idx_cp(b, b):
                c.start()
        for b in range(NB):
            step(b, b, b >= 2, b == NB - 1)

        def body(jj, carry):
            j = NB * jj
            for b in range(NB):
                step(j + b, b, True, True)
            return carry

        lax.fori_loop(1, nch // NB - 1, body, 0)
        for b in range(NB):
            step(nch - NB + b, b, True, b < NB - 1)
        for off in (2, 1):
            b = (nch - off) % NB
            g_cp(b).wait()
            pltpu.async_copy(rows.at[b], acc.at[dstc.at[b]], ss[b], add=True)
        for b in range(NB):
            sc_cp(b).wait()
        plsc.subcore_barrier()
        pltpu.sync_copy(acc.at[pl.ds(sid * rpt, rpt)],
                        out_hbm.at[cid].at[pl.ds(sid * rpt, rpt)])

    return prop


def _stage_a(omega_r, w0_r, b0_r, m0_r, w1_r, hc0_r, hc1_r, g1_r, dinv_r):
    h0 = jnp.dot(omega_r[...], w0_r[...], preferred_element_type=jnp.float32)
    h0 = jnp.maximum(h0 + b0_r[...], 0.0)
    h0 = jnp.where(m0_r[...], h0 * 2.0, 0.0)
    t1 = jnp.dot(h0, w1_r[...], preferred_element_type=jnp.float32)
    deg = hc0_r[...] + hc1_r[...] + 1.0
    dinv = lax.rsqrt(jnp.maximum(deg, 1.0))
    g1 = t1 * dinv
    F = t1.shape[1] // 2
    g1_r[0, :, :] = g1[:, :F]
    g1_r[1, :, :] = g1[:, F:]
    dinv_r[...] = dinv


def _stage_b(agg_r, g1_r, dinv_r, b1_r, m1_r, w2_r, g2_r):
    pre = agg_r[...] + g1_r[...]
    h1 = jnp.concatenate([pre[0], pre[1]], axis=1)
    dinv = dinv_r[...]
    h1 = h1 * dinv + b1_r[...]
    h1 = jnp.where(m1_r[...], jnp.maximum(h1, 0.0) * 2.0, 0.0)
    t2 = jnp.dot(h1, w2_r[...], preferred_element_type=jnp.float32)
    g2_r[...] = t2 * dinv


def _stage_c(agg_r, g2_r, dinv_r, b2_r, z_r, out_r):
    agg = agg_r[...]
    h2 = agg[0] + agg[1] + g2_r[...]
    out_r[...] = z_r[...] * (h2 * dinv_r[...] + b2_r[...])


def kernel(z, omega, edge_index, W0, b0, W1, b1, W2, b2):
    N, D_out = z.shape
    D_hid = W0.shape[1]
    E = edge_index.shape[1]
    R = 400
    grid = N // R
    NP = -(-N // (8 * NS)) * 8 * NS

    src = edge_index[0]
    dst = edge_index[1]

    dk = jax.random.key(42)
    m0 = jax.random.bernoulli(jax.random.fold_in(dk, 0), 0.5, (N, D_hid))
    m1 = jax.random.bernoulli(jax.random.fold_in(dk, 1), 0.5, (N, D_hid))

    zeros128 = jnp.zeros((NP, D_out), jnp.float32)
    ones128 = jnp.ones((K, D_out), jnp.float32)

    ept1 = -(-(E // NS) // (NB * K)) * NB * K
    s1 = jnp.pad(src.reshape(NS, E // NS), ((0, 0), (0, ept1 - E // NS)))
    src1 = jnp.stack([s1, s1 + N]).reshape(-1)
    dst1 = jnp.pad(dst.reshape(NS, E // NS), ((0, 0), (0, ept1 - E // NS)),
                   constant_values=N).reshape(-1)
    epw = E // (NC * NS)
    ept2 = -(-epw // (NB * K)) * NB * K
    src2 = jnp.pad(src.reshape(NC * NS, epw),
                   ((0, 0), (0, ept2 - epw))).reshape(-1)
    dst2 = jnp.pad(dst.reshape(NC * NS, epw), ((0, 0), (0, ept2 - epw)),
                   constant_values=N).reshape(-1)

    hist = _make_hist(N, NP, ept2)(dst2, ones128, zeros128)
    hc0 = hist[0, :, 0].reshape(NP, 1)
    hc1 = hist[1, :, 0].reshape(NP, 1)

    g1, dinv = pl.pallas_call(
        _stage_a,
        grid=(grid,),
        in_specs=[
            pl.BlockSpec((R, omega.shape[1]), lambda i: (i, 0)),
            pl.BlockSpec(W0.shape, lambda i: (0, 0)),
            pl.BlockSpec((1, D_hid), lambda i: (0, 0)),
            pl.BlockSpec((R, D_hid), lambda i: (i, 0)),
            pl.BlockSpec(W1.shape, lambda i: (0, 0)),
            pl.BlockSpec((R, 1), lambda i: (i, 0)),
            pl.BlockSpec((R, 1), lambda i: (i, 0)),
        ],
        out_specs=[
            pl.BlockSpec((NC, R, D_hid // 2), lambda i: (0, i, 0)),
            pl.BlockSpec((R, 1), lambda i: (i, 0)),
        ],
        out_shape=[
            jax.ShapeDtypeStruct((NC, N, D_hid // 2), jnp.float32),
            jax.ShapeDtypeStruct((N, 1), jnp.float32),
        ],
    )(omega, W0, b0.reshape(1, -1), m0, W1, hc0, hc1)

    agg1 = _make_prop(N, NP, src_stride=NS * ept1, dst_stride=0, ept=ept1)(
        g1.reshape(NC * N, D_hid // 2), src1, dst1, zeros128)

    g2 = pl.pallas_call(
        _stage_b,
        grid=(grid,),
        in_specs=[
            pl.BlockSpec((NC, R, D_hid // 2), lambda i: (0, i, 0)),
            pl.BlockSpec((NC, R, D_hid // 2), lambda i: (0, i, 0)),
            pl.BlockSpec((R, 1), lambda i: (i, 0)),
            pl.BlockSpec((1, D_hid), lambda i: (0, 0)),
            pl.BlockSpec((R, D_hid), lambda i: (i, 0)),
            pl.BlockSpec(W2.shape, lambda i: (0, 0)),
        ],
        out_specs=pl.BlockSpec((R, D_out), lambda i: (i, 0)),
        out_shape=jax.ShapeDtypeStruct((N, D_out), jnp.float32),
    )(agg1, g1, dinv, b1.reshape(1, -1), m1, W2)

    agg2 = _make_prop(N, NP, src_stride=NS * ept2, dst_stride=NS * ept2,
                      ept=ept2)(g2, src2, dst2, zeros128)

    out = pl.pallas_call(
        _stage_c,
        grid=(grid,),
        in_specs=[
            pl.BlockSpec((NC, R, D_out), lambda i: (0, i, 0)),
            pl.BlockSpec((R, D_out), lambda i: (i, 0)),
            pl.BlockSpec((R, 1), lambda i: (i, 0)),
            pl.BlockSpec((1, D_out), lambda i: (0, 0)),
            pl.BlockSpec((R, D_out), lambda i: (i, 0)),
        ],
        out_specs=pl.BlockSpec((R, D_out), lambda i: (i, 0)),
        out_shape=jax.ShapeDtypeStruct((N, D_out), jnp.float32),
    )(agg2, g2, dinv, b2.reshape(1, -1), z)

    return out


# speedup vs baseline: 1.0501x; 1.0501x over previous
import functools

import jax
import jax.numpy as jnp
from jax import lax
from jax.experimental import pallas as pl
from jax.experimental.pallas import tpu as pltpu
from jax.experimental.pallas import tpu_sc as plsc

NC = 2
NS = 16
K = 96
NB = 4


def _mesh():
    return plsc.VectorSubcoreMesh(core_axis_name="c", subcore_axis_name="s")


def _make_hist(N, NP, ept, F=128):
    nch = ept // K
    rpt = NP // NS
    assert nch % NB == 0 and nch >= 2 * NB

    @functools.partial(
        pl.kernel,
        out_type=jax.ShapeDtypeStruct((NC, NP, F), jnp.float32),
        mesh=_mesh(),
        scratch_types=[
            pltpu.VMEM((NB, K), jnp.int32),
            pltpu.VMEM((K, F), jnp.float32),
            pltpu.VMEM_SHARED((NP, F), jnp.float32),
        ] + [pltpu.SemaphoreType.DMA] * (2 * NB),
    )
    def hist(dst_hbm, ones_hbm, zeros_hbm, out_hbm, dstc, ones_v, acc, *sems):
        si = sems[:NB]
        ss = sems[NB:]
        cid = lax.axis_index("c")
        sid = lax.axis_index("s")
        base = (cid * NS + sid) * ept
        pltpu.sync_copy(zeros_hbm.at[pl.ds(sid * rpt, rpt)],
                        acc.at[pl.ds(sid * rpt, rpt)])
        pltpu.sync_copy(ones_hbm, ones_v)
        plsc.subcore_barrier()

        def idx_cp(j, b):
            return pltpu.make_async_copy(
                dst_hbm.at[pl.ds(base + j * K, K)], dstc.at[b], si[b])

        def sc_cp(b):
            return pltpu.make_async_copy(ones_v, acc.at[dstc.at[b]], ss[b])

        def step(j, b, issue):
            nb = (b + 1) % NB
            idx_cp(j, b).wait()
            pltpu.async_copy(ones_v, acc.at[dstc.at[b]], ss[b], add=True)
            if issue:
                sc_cp(nb).wait()
                idx_cp(j + 1, nb).start()

        for b in range(NB):
            idx_cp(b, b).start()
        for b in range(NB):
            step(b, b, b == NB - 1)

        def body(jj, carry):
            j = NB * jj
            for b in range(NB):
                step(j + b, b, True)
            return carry

        lax.fori_loop(1, nch // NB - 1, body, 0)
        for b in range(NB):
            step(nch - NB + b, b, b < NB - 1)
        for b in range(NB):
            sc_cp(b).wait()
        plsc.subcore_barrier()
        pltpu.sync_copy(acc.at[pl.ds(sid * rpt, rpt)],
                        out_hbm.at[cid].at[pl.ds(sid * rpt, rpt)])

    return hist


def _make_prop(N, NP, src_stride, dst_stride, ept, F=128):
    nch = ept // K
    rpt = NP // NS
    assert nch % NB == 0 and nch >= 2 * NB

    @functools.partial(
        pl.kernel,
        out_type=jax.ShapeDtypeStruct((NC, NP, F), jnp.float32),
        mesh=_mesh(),
        scratch_types=[
            pltpu.VMEM((NB, K), jnp.int32),
            pltpu.VMEM((NB, K), jnp.int32),
            pltpu.VMEM((NB, K, F), jnp.float32),
            pltpu.VMEM_SHARED((NP, F), jnp.float32),
        ] + [pltpu.SemaphoreType.DMA] * (3 * NB),
    )
    def prop(table_hbm, src_hbm, dst_hbm, zeros_hbm, out_hbm,
             srcc, dstc, rows, acc, *sems):
        si = sems[:NB]
        sg = sems[NB:2 * NB]
        ss = sems[2 * NB:]
        cid = lax.axis_index("c")
        sid = lax.axis_index("s")
        base_src = cid * src_stride + sid * ept
        base_dst = cid * dst_stride + sid * ept
        pltpu.sync_copy(zeros_hbm.at[pl.ds(sid * rpt, rpt)],
                        acc.at[pl.ds(sid * rpt, rpt)])
        plsc.subcore_barrier()

        def idx_cp(j, b):
            return (
                pltpu.make_async_copy(
                    src_hbm.at[pl.ds(base_src + j * K, K)], srcc.at[b], si[b]),
                pltpu.make_async_copy(
                    dst_hbm.at[pl.ds(base_dst + j * K, K)], dstc.at[b], si[b]))

        def g_cp(b):
            return pltpu.make_async_copy(table_hbm.at[srcc.at[b]],
                                         rows.at[b], sg[b])

        def sc_cp(b):
            return pltpu.make_async_copy(rows.at[b], acc.at[dstc.at[b]], ss[b])

        def step(j, b, scatter, issue):
            p2 = (b - 2) % NB
            nb = (b + 1) % NB
            for c in idx_cp(j, b):
                c.wait()
            g_cp(b).start()
            if scatter:
                g_cp(p2).wait()
                pltpu.async_copy(rows.at[p2], acc.at[dstc.at[p2]], ss[p2],
                                 add=True)
            if issue:
                sc_cp(nb).wait()
                for c in idx_cp(j + 1, nb):
                    c.start()

        for b in range(NB):
            for c in idx_cp(b, b):
                c.start()
        for b in range(NB):
            step(b, b, b >= 2, b == NB - 1)

        def body(jj, carry):
            j = NB * jj
            for b in range(NB):
                step(j + b, b, True, True)
            return carry

        lax.fori_loop(1, nch // NB - 1, body, 0)
        for b in range(NB):
            step(nch - NB + b, b, True, b < NB - 1)
        for off in (2, 1):
            b = (nch - off) % NB
            g_cp(b).wait()
            pltpu.async_copy(rows.at[b], acc.at[dstc.at[b]], ss[b], add=True)
        for b in range(NB):
            sc_cp(b).wait()
        plsc.subcore_barrier()
        pltpu.sync_copy(acc.at[pl.ds(sid * rpt, rpt)],
                        out_hbm.at[cid].at[pl.ds(sid * rpt, rpt)])

    return prop


def _stage_a(omega_r, w0_r, b0_r, m0_r, w1_r, hc0_r, hc1_r, g1_r, dinv_r):
    h0 = jnp.dot(omega_r[...], w0_r[...], preferred_element_type=jnp.float32)
    h0 = jnp.maximum(h0 + b0_r[...], 0.0)
    h0 = jnp.where(m0_r[...], h0 * 2.0, 0.0)
    t1 = jnp.dot(h0, w1_r[...], preferred_element_type=jnp.float32)
    deg = hc0_r[...] + hc1_r[...] + 1.0
    dinv = lax.rsqrt(jnp.maximum(deg, 1.0))
    g1 = t1 * dinv
    F = t1.shape[1] // 2
    g1_r[0, :, :] = g1[:, :F]
    g1_r[1, :, :] = g1[:, F:]
    dinv_r[...] = dinv


def _stage_b(agg_r, g1_r, dinv_r, b1_r, m1_r, w2_r, g2_r):
    pre = agg_r[...] + g1_r[...]
    h1 = jnp.concatenate([pre[0], pre[1]], axis=1)
    dinv = dinv_r[...]
    h1 = h1 * dinv + b1_r[...]
    h1 = jnp.where(m1_r[...], jnp.maximum(h1, 0.0) * 2.0, 0.0)
    t2 = jnp.dot(h1, w2_r[...], preferred_element_type=jnp.float32)
    g2_r[...] = t2 * dinv


def _stage_c(agg_r, g2_r, dinv_r, b2_r, z_r, out_r):
    agg = agg_r[...]
    h2 = agg[0] + agg[1] + g2_r[...]
    out_r[...] = z_r[...] * (h2 * dinv_r[...] + b2_r[...])


def kernel(z, omega, edge_index, W0, b0, W1, b1, W2, b2):
    N, D_out = z.shape
    D_hid = W0.shape[1]
    E = edge_index.shape[1]
    R = 400
    grid = N // R
    NP = -(-N // (8 * NS)) * 8 * NS

    src = edge_index[0]
    dst = edge_index[1]

    dk = jax.random.key(42)
    m0 = jax.random.bernoulli(jax.random.fold_in(dk, 0), 0.5, (N, D_hid))
    m1 = jax.random.bernoulli(jax.random.fold_in(dk, 1), 0.5, (N, D_hid))

    zeros128 = jnp.zeros((NP, D_out), jnp.float32)
    ones128 = jnp.ones((K, D_out), jnp.float32)

    ept1 = -(-(E // NS) // (NB * K)) * NB * K
    s1 = jnp.pad(src.reshape(NS, E // NS), ((0, 0), (0, ept1 - E // NS)))
    src1 = jnp.stack([s1, s1 + N]).reshape(-1)
    dst1 = jnp.pad(dst.reshape(NS, E // NS), ((0, 0), (0, ept1 - E // NS)),
                   constant_values=N).reshape(-1)
    epw = E // (NC * NS)
    ept2 = -(-epw // (NB * K)) * NB * K
    src2 = jnp.pad(src.reshape(NC * NS, epw),
                   ((0, 0), (0, ept2 - epw))).reshape(-1)
    dst2 = jnp.pad(dst.reshape(NC * NS, epw), ((0, 0), (0, ept2 - epw)),
                   constant_values=N).reshape(-1)

    hist = _make_hist(N, NP, ept2)(dst2, ones128, zeros128)
    hc0 = hist[0, :, 0].reshape(NP, 1)
    hc1 = hist[1, :, 0].reshape(NP, 1)

    g1, dinv = pl.pallas_call(
        _stage_a,
        grid=(grid,),
        in_specs=[
            pl.BlockSpec((R, omega.shape[1]), lambda i: (i, 0)),
            pl.BlockSpec(W0.shape, lambda i: (0, 0)),
            pl.BlockSpec((1, D_hid), lambda i: (0, 0)),
            pl.BlockSpec((R, D_hid), lambda i: (i, 0)),
            pl.BlockSpec(W1.shape, lambda i: (0, 0)),
            pl.BlockSpec((R, 1), lambda i: (i, 0)),
            pl.BlockSpec((R, 1), lambda i: (i, 0)),
        ],
        out_specs=[
            pl.BlockSpec((NC, R, D_hid // 2), lambda i: (0, i, 0)),
            pl.BlockSpec((R, 1), lambda i: (i, 0)),
        ],
        out_shape=[
            jax.ShapeDtypeStruct((NC, N, D_hid // 2), jnp.float32),
            jax.ShapeDtypeStruct((N, 1), jnp.float32),
        ],
    )(omega, W0, b0.reshape(1, -1), m0, W1, hc0, hc1)

    agg1 = _make_prop(N, NP, src_stride=NS * ept1, dst_stride=0, ept=ept1)(
        g1.reshape(NC * N, D_hid // 2), src1, dst1, zeros128)

    g2 = pl.pallas_call(
        _stage_b,
        grid=(grid,),
        in_specs=[
            pl.BlockSpec((NC, R, D_hid // 2), lambda i: (0, i, 0)),
            pl.BlockSpec((NC, R, D_hid // 2), lambda i: (0, i, 0)),
            pl.BlockSpec((R, 1), lambda i: (i, 0)),
            pl.BlockSpec((1, D_hid), lambda i: (0, 0)),
            pl.BlockSpec((R, D_hid), lambda i: (i, 0)),
            pl.BlockSpec(W2.shape, lambda i: (0, 0)),
        ],
        out_specs=pl.BlockSpec((R, D_out), lambda i: (i, 0)),
        out_shape=jax.ShapeDtypeStruct((N, D_out), jnp.float32),
    )(agg1, g1, dinv, b1.reshape(1, -1), m1, W2)

    agg2 = _make_prop(N, NP, src_stride=NS * ept2, dst_stride=NS * ept2,
                      ept=ept2)(g2, src2, dst2, zeros128)

    out = pl.pallas_call(
        _stage_c,
        grid=(grid,),
        in_specs=[
            pl.BlockSpec((NC, R, D_out), lambda i: (0, i, 0)),
            pl.BlockSpec((R, D_out), lambda i: (i, 0)),
            pl.BlockSpec((R, 1), lambda i: (i, 0)),
            pl.BlockSpec((1, D_out), lambda i: (0, 0)),
            pl.BlockSpec((R, D_out), lambda i: (i, 0)),
        ],
        out_specs=pl.BlockSpec((R, D_out), lambda i: (i, 0)),
        out_shape=jax.ShapeDtypeStruct((N, D_out), jnp.float32),
    )(agg2, g2, dinv, b2.reshape(1, -1), z)

    return out


# speedup vs baseline: 1.6082x; 1.5315x over previous
import functools

import jax
import jax.numpy as jnp
from jax import lax
from jax.experimental import pallas as pl
from jax.experimental.pallas import tpu as pltpu
from jax.experimental.pallas import tpu_sc as plsc

NC = 2
NS = 16
K = 112
NB = 3


def _mesh():
    return plsc.VectorSubcoreMesh(core_axis_name="c", subcore_axis_name="s")


def _make_hist(N, NP, ept, F=128):
    nch = ept // K
    rpt = NP // NS
    assert nch % NB == 0 and nch >= 2 * NB

    @functools.partial(
        pl.kernel,
        out_type=jax.ShapeDtypeStruct((NC, NP, F), jnp.float32),
        mesh=_mesh(),
        scratch_types=[
            pltpu.VMEM((NB, K), jnp.int32),
            pltpu.VMEM((K, F), jnp.float32),
            pltpu.VMEM_SHARED((NP, F), jnp.float32),
        ] + [pltpu.SemaphoreType.DMA] * (2 * NB),
    )
    def hist(dst_hbm, ones_hbm, zeros_hbm, out_hbm, dstc, ones_v, acc, *sems):
        si = sems[:NB]
        ss = sems[NB:]
        cid = lax.axis_index("c")
        sid = lax.axis_index("s")
        base = (cid * NS + sid) * ept
        pltpu.sync_copy(zeros_hbm.at[pl.ds(sid * rpt, rpt)],
                        acc.at[pl.ds(sid * rpt, rpt)])
        pltpu.sync_copy(ones_hbm, ones_v)
        plsc.subcore_barrier()

        def idx_cp(j, b):
            return pltpu.make_async_copy(
                dst_hbm.at[pl.ds(base + j * K, K)], dstc.at[b], si[b])

        def sc_cp(b):
            return pltpu.make_async_copy(ones_v, acc.at[dstc.at[b]], ss[b])

        def step(j, b, issue):
            nb = (b + 1) % NB
            idx_cp(j, b).wait()
            pltpu.async_copy(ones_v, acc.at[dstc.at[b]], ss[b], add=True)
            if issue:
                sc_cp(nb).wait()
                idx_cp(j + 1, nb).start()

        for b in range(NB):
            idx_cp(b, b).start()
        for b in range(NB):
            step(b, b, b == NB - 1)

        def body(jj, carry):
            j = NB * jj
            for b in range(NB):
                step(j + b, b, True)
            return carry

        lax.fori_loop(1, nch // NB - 1, body, 0)
        for b in range(NB):
            step(nch - NB + b, b, b < NB - 1)
        for b in range(NB):
            sc_cp(b).wait()
        plsc.subcore_barrier()
        pltpu.sync_copy(acc.at[pl.ds(sid * rpt, rpt)],
                        out_hbm.at[cid].at[pl.ds(sid * rpt, rpt)])

    return hist


def _make_prop(N, NP, src_stride, dst_stride, ept, F=128):
    nch = ept // K
    rpt = NP // NS
    assert nch % NB == 0 and nch >= 2 * NB

    @functools.partial(
        pl.kernel,
        out_type=jax.ShapeDtypeStruct((NC, NP, F), jnp.float32),
        mesh=_mesh(),
        scratch_types=[
            pltpu.VMEM((NB, K), jnp.int32),
            pltpu.VMEM((NB, K), jnp.int32),
            pltpu.VMEM((NB, K, F), jnp.float32),
            pltpu.VMEM_SHARED((NP, F), jnp.float32),
        ] + [pltpu.SemaphoreType.DMA] * (3 * NB),
    )
    def prop(table_hbm, src_hbm, dst_hbm, zeros_hbm, out_hbm,
             srcc, dstc, rows, acc, *sems):
        si = sems[:NB]
        sg = sems[NB:2 * NB]
        ss = sems[2 * NB:]
        cid = lax.axis_index("c")
        sid = lax.axis_index("s")
        base_src = cid * src_stride + sid * ept
        base_dst = cid * dst_stride + sid * ept
        pltpu.sync_copy(zeros_hbm.at[pl.ds(sid * rpt, rpt)],
                        acc.at[pl.ds(sid * rpt, rpt)])
        plsc.subcore_barrier()

        def idx_cp(j, b):
            return (
                pltpu.make_async_copy(
                    src_hbm.at[pl.ds(base_src + j * K, K)], srcc.at[b], si[b]),
                pltpu.make_async_copy(
                    dst_hbm.at[pl.ds(base_dst + j * K, K)], dstc.at[b], si[b]))

        def g_cp(b):
            return pltpu.make_async_copy(table_hbm.at[srcc.at[b]],
                                         rows.at[b], sg[b])

        def sc_cp(b):
            return pltpu.make_async_copy(rows.at[b], acc.at[dstc.at[b]], ss[b])

        def step(j, b, scatter, issue):
            p2 = (b - 2) % NB
            nb = (b + 1) % NB
            for c in idx_cp(j, b):
                c.wait()
            g_cp(b).start()
            if scatter:
                g_cp(p2).wait()
                pltpu.async_copy(rows.at[p2], acc.at[dstc.at[p2]], ss[p2],
                                 add=True)
            if issue:
                sc_cp(nb).wait()
                for c in idx_cp(j + 1, nb):
                    c.start()

        for b in range(NB):
            for c in idx_cp(b, b):
                c.start()
        for b in range(NB):
            step(b, b, b >= 2, b == NB - 1)

        def body(jj, carry):
            j = NB * jj
            for b in range(NB):
                step(j + b, b, True, True)
            return carry

        lax.fori_loop(1, nch // NB - 1, body, 0)
        for b in range(NB):
            step(nch - NB + b, b, True, b < NB - 1)
        for off in (2, 1):
            b = (nch - off) % NB
            g_cp(b).wait()
            pltpu.async_copy(rows.at[b], acc.at[dstc.at[b]], ss[b], add=True)
        for b in range(NB):
            sc_cp(b).wait()
        plsc.subcore_barrier()
        pltpu.sync_copy(acc.at[pl.ds(sid * rpt, rpt)],
                        out_hbm.at[cid].at[pl.ds(sid * rpt, rpt)])

    return prop


def _stage_a(omega_r, w0_r, b0_r, m0_r, w1_r, hc0_r, hc1_r, g1_r, dinv_r):
    h0 = jnp.dot(omega_r[...], w0_r[...], preferred_element_type=jnp.float32)
    h0 = jnp.maximum(h0 + b0_r[...], 0.0)
    h0 = jnp.where(m0_r[...], h0 * 2.0, 0.0)
    t1 = jnp.dot(h0, w1_r[...], preferred_element_type=jnp.float32)
    deg = hc0_r[...] + hc1_r[...] + 1.0
    dinv = lax.rsqrt(jnp.maximum(deg, 1.0))
    g1 = t1 * dinv
    F = t1.shape[1] // 2
    g1_r[0, :, :] = g1[:, :F]
    g1_r[1, :, :] = g1[:, F:]
    dinv_r[...] = dinv


def _stage_b(agg_r, g1_r, dinv_r, b1_r, m1_r, w2_r, g2_r):
    pre = agg_r[...] + g1_r[...]
    h1 = jnp.concatenate([pre[0], pre[1]], axis=1)
    dinv = dinv_r[...]
    h1 = h1 * dinv + b1_r[...]
    h1 = jnp.where(m1_r[...], jnp.maximum(h1, 0.0) * 2.0, 0.0)
    t2 = jnp.dot(h1, w2_r[...], preferred_element_type=jnp.float32)
    g2_r[...] = t2 * dinv


def _stage_c(agg_r, g2_r, dinv_r, b2_r, z_r, out_r):
    agg = agg_r[...]
    h2 = agg[0] + agg[1] + g2_r[...]
    out_r[...] = z_r[...] * (h2 * dinv_r[...] + b2_r[...])


def kernel(z, omega, edge_index, W0, b0, W1, b1, W2, b2):
    N, D_out = z.shape
    D_hid = W0.shape[1]
    E = edge_index.shape[1]
    R = 400
    grid = N // R
    NP = -(-N // (8 * NS)) * 8 * NS

    src = edge_index[0]
    dst = edge_index[1]

    dk = jax.random.key(42)
    m0 = jax.random.bernoulli(jax.random.fold_in(dk, 0), 0.5, (N, D_hid))
    m1 = jax.random.bernoulli(jax.random.fold_in(dk, 1), 0.5, (N, D_hid))

    zeros128 = jnp.zeros((NP, D_out), jnp.float32)
    ones128 = jnp.ones((K, D_out), jnp.float32)

    ept1 = -(-(E // NS) // (NB * K)) * NB * K
    s1 = jnp.pad(src.reshape(NS, E // NS), ((0, 0), (0, ept1 - E // NS)))
    src1 = jnp.stack([s1, s1 + N]).reshape(-1)
    dst1 = jnp.pad(dst.reshape(NS, E // NS), ((0, 0), (0, ept1 - E // NS)),
                   constant_values=N).reshape(-1)
    epw = E // (NC * NS)
    ept2 = -(-epw // (NB * K)) * NB * K
    src2 = jnp.pad(src.reshape(NC * NS, epw),
                   ((0, 0), (0, ept2 - epw))).reshape(-1)
    dst2 = jnp.pad(dst.reshape(NC * NS, epw), ((0, 0), (0, ept2 - epw)),
                   constant_values=N).reshape(-1)

    hist = _make_hist(N, NP, ept2)(dst2, ones128, zeros128)
    hc0 = hist[0, :, 0].reshape(NP, 1)
    hc1 = hist[1, :, 0].reshape(NP, 1)

    g1, dinv = pl.pallas_call(
        _stage_a,
        grid=(grid,),
        in_specs=[
            pl.BlockSpec((R, omega.shape[1]), lambda i: (i, 0)),
            pl.BlockSpec(W0.shape, lambda i: (0, 0)),
            pl.BlockSpec((1, D_hid), lambda i: (0, 0)),
            pl.BlockSpec((R, D_hid), lambda i: (i, 0)),
            pl.BlockSpec(W1.shape, lambda i: (0, 0)),
            pl.BlockSpec((R, 1), lambda i: (i, 0)),
            pl.BlockSpec((R, 1), lambda i: (i, 0)),
        ],
        out_specs=[
            pl.BlockSpec((NC, R, D_hid // 2), lambda i: (0, i, 0)),
            pl.BlockSpec((R, 1), lambda i: (i, 0)),
        ],
        out_shape=[
            jax.ShapeDtypeStruct((NC, N, D_hid // 2), jnp.float32),
            jax.ShapeDtypeStruct((N, 1), jnp.float32),
        ],
    )(omega, W0, b0.reshape(1, -1), m0, W1, hc0, hc1)

    agg1 = _make_prop(N, NP, src_stride=NS * ept1, dst_stride=0, ept=ept1)(
        g1.reshape(NC * N, D_hid // 2), src1, dst1, zeros128)

    g2 = pl.pallas_call(
        _stage_b,
        grid=(grid,),
        in_specs=[
            pl.BlockSpec((NC, R, D_hid // 2), lambda i: (0, i, 0)),
            pl.BlockSpec((NC, R, D_hid // 2), lambda i: (0, i, 0)),
            pl.BlockSpec((R, 1), lambda i: (i, 0)),
            pl.BlockSpec((1, D_hid), lambda i: (0, 0)),
            pl.BlockSpec((R, D_hid), lambda i: (i, 0)),
            pl.BlockSpec(W2.shape, lambda i: (0, 0)),
        ],
        out_specs=pl.BlockSpec((R, D_out), lambda i: (i, 0)),
        out_shape=jax.ShapeDtypeStruct((N, D_out), jnp.float32),
    )(agg1, g1, dinv, b1.reshape(1, -1), m1, W2)

    agg2 = _make_prop(N, NP, src_stride=NS * ept2, dst_stride=NS * ept2,
                      ept=ept2)(g2, src2, dst2, zeros128)

    out = pl.pallas_call(
        _stage_c,
        grid=(grid,),
        in_specs=[
            pl.BlockSpec((NC, R, D_out), lambda i: (0, i, 0)),
            pl.BlockSpec((R, D_out), lambda i: (i, 0)),
            pl.BlockSpec((R, 1), lambda i: (i, 0)),
            pl.BlockSpec((1, D_out), lambda i: (0, 0)),
            pl.BlockSpec((R, D_out), lambda i: (i, 0)),
        ],
        out_specs=pl.BlockSpec((R, D_out), lambda i: (i, 0)),
        out_shape=jax.ShapeDtypeStruct((N, D_out), jnp.float32),
    )(agg2, g2, dinv, b2.reshape(1, -1), z)

    return out
